# Initial kernel scaffold; baseline (speedup 1.0000x reference)
import functools

import jax
import jax.numpy as jnp
from jax import lax
from jax.experimental import pallas as pl
from jax.experimental.pallas import tpu as pltpu
from jax.experimental.pallas import tpu_sc as plsc

B, T = 4, 8192
VOCAB, CHAR_DIM, N_FREQ = 1000, 32, 4
OUT_DIM = CHAR_DIM * N_FREQ
NTOK = B * T
NUM_CORES, NUM_SUBCORES, LANES = 2, 16, 16
NW = NUM_CORES * NUM_SUBCORES
TPW = NTOK // NW
CHUNK = 128
NCH = TPW // CHUNK

_mesh = plsc.VectorSubcoreMesh(core_axis_name="c", subcore_axis_name="s")


@functools.partial(
    pl.kernel,
    out_type=jax.ShapeDtypeStruct((NTOK, OUT_DIM), jnp.float32),
    mesh=_mesh,
    scratch_types=[
        pltpu.VMEM((CHUNK,), jnp.int32),
        pltpu.VMEM((CHUNK, OUT_DIM), jnp.float32),
        [pltpu.VMEM((CHUNK, CHAR_DIM), jnp.float32) for _ in range(N_FREQ)],
        pltpu.SemaphoreType.DMA,
    ],
)
def _encode(idx_hbm, emb0, emb1, emb2, emb3, pos_hbm, out_hbm,
            idx_v, acc_v, rows_v, sem):
    tables = (emb0, emb1, emb2, emb3)
    w = lax.axis_index("s") * NUM_CORES + lax.axis_index("c")
    base = w * TPW

    def chunk_body(ch, carry):
        tok0 = base + ch * CHUNK
        pos0 = lax.rem(tok0, T)
        pltpu.sync_copy(idx_hbm.at[pl.ds(tok0, CHUNK)], idx_v)
        pltpu.sync_copy(pos_hbm.at[pl.ds(pos0, CHUNK)], acc_v)
        descs = [pltpu.async_copy(tables[c].at[idx_v], rows_v[c], sem)
                 for c in range(N_FREQ)]
        for d in descs:
            d.wait()

        def tok_body(i, c2):
            for c in range(N_FREQ):
                for k in range(CHAR_DIM // LANES):
                    v = rows_v[c][i, pl.ds(k * LANES, LANES)]
                    plsc.addupdate(
                        acc_v.at[i, pl.ds(c * CHAR_DIM + k * LANES, LANES)], v)
            return c2

        lax.fori_loop(0, CHUNK, tok_body, 0)
        pltpu.sync_copy(acc_v, out_hbm.at[pl.ds(tok0, CHUNK)])
        return carry

    lax.fori_loop(0, NCH, chunk_body, 0)


def kernel(raw_char_indices, emb0, emb1, emb2, emb3, pos_table):
    idx = raw_char_indices.reshape(NTOK)
    out = _encode(idx, emb0, emb1, emb2, emb3, pos_table)
    return out.reshape(B, T, OUT_DIM)


# speedup vs baseline: 8.0020x; 8.0020x over previous
import functools

import jax
import jax.numpy as jnp
from jax import lax
from jax.experimental import pallas as pl
from jax.experimental.pallas import tpu as pltpu
from jax.experimental.pallas import tpu_sc as plsc

B, T = 4, 8192
VOCAB, CHAR_DIM, N_FREQ = 1000, 32, 4
OUT_DIM = CHAR_DIM * N_FREQ
NTOK = B * T
NUM_CORES, NUM_SUBCORES, LANES = 2, 16, 16
NW = NUM_CORES * NUM_SUBCORES
TPW = NTOK // NW
CHUNK = 128
NCH = TPW // CHUNK

_mesh = plsc.VectorSubcoreMesh(core_axis_name="c", subcore_axis_name="s")


@functools.partial(
    pl.kernel,
    out_type=jax.ShapeDtypeStruct((NTOK, OUT_DIM), jnp.float32),
    mesh=_mesh,
    scratch_types=[
        pltpu.VMEM((CHUNK,), jnp.int32),
        pltpu.VMEM((CHUNK, OUT_DIM), jnp.float32),
        pltpu.VMEM((CHUNK, OUT_DIM), jnp.float32),
        pltpu.SemaphoreType.DMA,
    ],
)
def _encode(idx_hbm, cat_hbm, pos_hbm, out_hbm, idx_v, acc_v, rows_v, sem):
    w = lax.axis_index("s") * NUM_CORES + lax.axis_index("c")
    base = w * TPW

    def chunk_body(ch, carry):
        tok0 = base + ch * CHUNK
        pos0 = lax.rem(tok0, T)
        pltpu.sync_copy(idx_hbm.at[pl.ds(tok0, CHUNK)], idx_v)
        pltpu.sync_copy(pos_hbm.at[pl.ds(pos0, CHUNK)], acc_v)
        pltpu.async_copy(cat_hbm.at[idx_v], rows_v, sem).wait()

        def tok_body(i, c2):
            for k in range(OUT_DIM // LANES):
                v = rows_v[i, pl.ds(k * LANES, LANES)]
                plsc.addupdate(acc_v.at[i, pl.ds(k * LANES, LANES)], v)
            return c2

        lax.fori_loop(0, CHUNK, tok_body, 0)
        pltpu.sync_copy(acc_v, out_hbm.at[pl.ds(tok0, CHUNK)])
        return carry

    lax.fori_loop(0, NCH, chunk_body, 0)


def kernel(raw_char_indices, emb0, emb1, emb2, emb3, pos_table):
    idx = raw_char_indices.reshape(NTOK)
    cat = jnp.concatenate([emb0, emb1, emb2, emb3], axis=1)
    out = _encode(idx, cat, pos_table)
    return out.reshape(B, T, OUT_DIM)


# speedup vs baseline: 10.9631x; 1.3700x over previous
import functools

import jax
import jax.numpy as jnp
from jax import lax
from jax.experimental import pallas as pl
from jax.experimental.pallas import tpu as pltpu
from jax.experimental.pallas import tpu_sc as plsc

B, T = 4, 8192
VOCAB, CHAR_DIM, N_FREQ = 1000, 32, 4
OUT_DIM = CHAR_DIM * N_FREQ
NTOK = B * T
NUM_CORES, NUM_SUBCORES, LANES = 2, 16, 16
NW = NUM_CORES * NUM_SUBCORES
TPW = NTOK // NW
CHUNK = 128
NCH = TPW // CHUNK

_mesh = plsc.VectorSubcoreMesh(core_axis_name="c", subcore_axis_name="s")


@functools.partial(
    pl.kernel,
    out_type=jax.ShapeDtypeStruct((NTOK, OUT_DIM), jnp.float32),
    mesh=_mesh,
    scratch_types=[
        [pltpu.VMEM((CHUNK,), jnp.int32) for _ in range(NCH)],
        [pltpu.VMEM((CHUNK, OUT_DIM), jnp.float32) for _ in range(2)],
        [pltpu.VMEM((CHUNK, OUT_DIM), jnp.float32) for _ in range(2)],
        [pltpu.SemaphoreType.DMA for _ in range(NCH)],
        [pltpu.SemaphoreType.DMA for _ in range(2)],
        [pltpu.SemaphoreType.DMA for _ in range(2)],
        [pltpu.SemaphoreType.DMA for _ in range(2)],
    ],
)
def _encode(idx_hbm, cat_hbm, pos_hbm, out_hbm,
            idx_v, acc_v, rows_v, si, sp, sg, so):
    w = lax.axis_index("s") * NUM_CORES + lax.axis_index("c")
    base = w * TPW

    def tok0_of(ch):
        return base + ch * CHUNK

    def start_idx(ch):
        return pltpu.async_copy(
            idx_hbm.at[pl.ds(tok0_of(ch), CHUNK)], idx_v[ch], si[ch])

    def start_pos(ch):
        p = ch % 2
        pos0 = lax.rem(tok0_of(ch), T)
        return pltpu.async_copy(pos_hbm.at[pl.ds(pos0, CHUNK)], acc_v[p], sp[p])

    def start_gather(ch):
        p = ch % 2
        return pltpu.async_copy(cat_hbm.at[idx_v[ch]], rows_v[p], sg[p])

    def start_out(ch):
        p = ch % 2
        return pltpu.async_copy(
            acc_v[p], out_hbm.at[pl.ds(tok0_of(ch), CHUNK)], so[p])

    di = [None] * NCH
    di[0] = start_idx(0)
    di[1] = start_idx(1)
    dpos = [start_pos(0), None]
    di[0].wait()
    dg = [start_gather(0), None]
    dout = [None, None]

    for ch in range(NCH):
        p = ch % 2
        q = 1 - p
        if ch + 1 < NCH:
            di[ch + 1].wait()
            dg[q] = start_gather(ch + 1)
            if dout[q] is not None:
                dout[q].wait()
                dout[q] = None
            dpos[q] = start_pos(ch + 1)
        if ch + 2 < NCH:
            di[ch + 2] = start_idx(ch + 2)
        dg[p].wait()
        dpos[p].wait()

        acc = acc_v[p]
        rows = rows_v[p]

        def tok_body(i, c2):
            for k in range(OUT_DIM // LANES):
                v = rows[i, pl.ds(k * LANES, LANES)]
                plsc.addupdate(acc.at[i, pl.ds(k * LANES, LANES)], v)
            return c2

        lax.fori_loop(0, CHUNK, tok_body, 0)

        dout[p] = start_out(ch)

    dout[0].wait()
    dout[1].wait()


def kernel(raw_char_indices, emb0, emb1, emb2, emb3, pos_table):
    idx = raw_char_indices.reshape(NTOK)
    cat = jnp.concatenate([emb0, emb1, emb2, emb3], axis=1)
    out = _encode(idx, cat, pos_table)
    return out.reshape(B, T, OUT_DIM)


# speedup vs baseline: 12.0917x; 1.1029x over previous
import functools

import jax
import jax.numpy as jnp
from jax import lax
from jax.experimental import pallas as pl
from jax.experimental.pallas import tpu as pltpu
from jax.experimental.pallas import tpu_sc as plsc

B, T = 4, 8192
VOCAB, CHAR_DIM, N_FREQ = 1000, 32, 4
OUT_DIM = CHAR_DIM * N_FREQ
NTOK = B * T
NUM_CORES, NUM_SUBCORES, LANES = 2, 16, 16
NW = NUM_CORES * NUM_SUBCORES
TPW = NTOK // NW
CHUNK = 128
NCH = TPW // CHUNK

_mesh = plsc.VectorSubcoreMesh(core_axis_name="c", subcore_axis_name="s")


@functools.partial(
    pl.kernel,
    out_type=jax.ShapeDtypeStruct((NTOK, OUT_DIM), jnp.float32),
    mesh=_mesh,
    scratch_types=[
        [pltpu.VMEM((CHUNK,), jnp.int32) for _ in range(NCH)],
        [pltpu.VMEM((CHUNK, OUT_DIM), jnp.float32) for _ in range(2)],
        [pltpu.VMEM((CHUNK, OUT_DIM), jnp.float32) for _ in range(2)],
        [pltpu.SemaphoreType.DMA for _ in range(NCH)],
        [pltpu.SemaphoreType.DMA for _ in range(2)],
        [pltpu.SemaphoreType.DMA for _ in range(2)],
        [pltpu.SemaphoreType.DMA for _ in range(2)],
        pltpu.VMEM_SHARED((VOCAB, OUT_DIM), jnp.float32),
    ],
)
def _encode(idx_hbm, cat_hbm, pos_hbm, out_hbm,
            idx_v, acc_v, rows_v, si, sp, sg, so, cat_sh):
    w = lax.axis_index("s") * NUM_CORES + lax.axis_index("c")
    base = w * TPW
    sid = lax.axis_index("s")

    r0 = jnp.minimum(sid * 64, VOCAB - 64)
    pltpu.sync_copy(cat_hbm.at[pl.ds(r0, 64)], cat_sh.at[pl.ds(r0, 64)])
    plsc.subcore_barrier()

    def tok0_of(ch):
        return base + ch * CHUNK

    def start_idx(ch):
        return pltpu.async_copy(
            idx_hbm.at[pl.ds(tok0_of(ch), CHUNK)], idx_v[ch], si[ch])

    def start_pos(ch):
        p = ch % 2
        pos0 = lax.rem(tok0_of(ch), T)
        return pltpu.async_copy(pos_hbm.at[pl.ds(pos0, CHUNK)], acc_v[p], sp[p])

    def start_gather(ch):
        p = ch % 2
        return pltpu.async_copy(cat_sh.at[idx_v[ch]], rows_v[p], sg[p])

    def start_out(ch):
        p = ch % 2
        return pltpu.async_copy(
            acc_v[p], out_hbm.at[pl.ds(tok0_of(ch), CHUNK)], so[p])

    di = [None] * NCH
    di[0] = start_idx(0)
    di[1] = start_idx(1)
    dpos = [start_pos(0), None]
    di[0].wait()
    dg = [start_gather(0), None]
    dout = [None, None]

    for ch in range(NCH):
        p = ch % 2
        q = 1 - p
        if ch + 1 < NCH:
            di[ch + 1].wait()
            dg[q] = start_gather(ch + 1)
            if dout[q] is not None:
                dout[q].wait()
                dout[q] = None
            dpos[q] = start_pos(ch + 1)
        if ch + 2 < NCH:
            di[ch + 2] = start_idx(ch + 2)
        dg[p].wait()
        dpos[p].wait()

        acc = acc_v[p]
        rows = rows_v[p]

        def tok_body(i, c2):
            for k in range(OUT_DIM // LANES):
                v = rows[i, pl.ds(k * LANES, LANES)]
                plsc.addupdate(acc.at[i, pl.ds(k * LANES, LANES)], v)
            return c2

        lax.fori_loop(0, CHUNK, tok_body, 0)

        dout[p] = start_out(ch)

    dout[0].wait()
    dout[1].wait()


def kernel(raw_char_indices, emb0, emb1, emb2, emb3, pos_table):
    idx = raw_char_indices.reshape(NTOK)
    cat = jnp.concatenate([emb0, emb1, emb2, emb3], axis=1)
    out = _encode(idx, cat, pos_table)
    return out.reshape(B, T, OUT_DIM)
